# baseline (speedup 1.0000x reference)
import functools

import jax
import jax.numpy as jnp
from jax import lax
from jax.experimental import pallas as pl
from jax.experimental.pallas import tpu as pltpu
from jax.experimental.pallas import tpu_sc as plsc

B = 16
MAX_LEN = 4096
D = 1024
N_TOK = 32768
NW = 32
NP = 64
PRW = (B * MAX_LEN) // NP
C = 48
ZC = 16
ZRING = 4


def _body(embed_hbm, imgf_hbm, tsrc_hbm, tdst_hbm, isrc_hbm, idst_hbm,
          meta_hbm, out_hbm, meta_v, ia0, id0, ia1, id1, rows0, rows1, zbuf,
          semz, semg0, semg1, sems0, sems1):
    cid = lax.axis_index("c")
    sid = lax.axis_index("s")
    wid = sid * 2 + cid

    pltpu.sync_copy(meta_hbm, meta_v)

    def zfill(k, carry):
        row = k // 8
        col0 = (k % 8) * 128
        for u in range(8):
            zbuf[row, pl.ds(col0 + u * 16, 16)] = jnp.zeros((16,), jnp.float32)
        return carry

    lax.fori_loop(0, (ZC * D) // 128, zfill, 0)

    q0 = 4 * (wid // 2) + (wid % 2)

    def piece(pi, carry):
        q = q0 + 2 * pi
        r0 = q * PRW
        mvec = meta_v[pl.ds(q * 16, 16)]
        t_lo = mvec[0]
        t_hi = mvec[1]
        i_lo = mvec[2]
        i_hi = mvec[3]
        pad_off = mvec[4]

        p8 = (pad_off // 8) * 8
        nz = PRW - p8
        nfull = nz // ZC

        def zdrain():
            pltpu.make_async_copy(out_hbm.at[pl.ds(r0, ZC)], zbuf, semz).wait()

        def zchunk(j, c2):
            start = r0 + PRW - (j + 1) * ZC
            pltpu.async_copy(zbuf, out_hbm.at[pl.ds(start, ZC)], semz)

            @pl.when(j >= ZRING)
            def _():
                zdrain()

            return c2

        lax.fori_loop(0, nfull, zchunk, 0)

        def zdrain_loop(j, c2):
            zdrain()
            return c2

        lax.fori_loop(0, jnp.minimum(nfull, ZRING), zdrain_loop, 0)

        h = nz - nfull * ZC

        @pl.when(h == 8)
        def _():
            pltpu.sync_copy(zbuf.at[pl.ds(0, 8)], out_hbm.at[pl.ds(r0 + p8, 8)])

        def gather_scatter(lo, hi, src_hbm, dst_hbm, table_hbm):
            lo8 = (lo // 8) * 8
            n = hi - lo8
            nch = (n + C - 1) // C

            def start_chunk(c, ia, idd, rows, semg, sems_):
                @pl.when(c < nch)
                def _():
                    @pl.when(c >= 2)
                    def _():
                        pltpu.make_async_copy(out_hbm.at[pl.ds(r0, C)], rows,
                                              sems_).wait()

                    off = jnp.minimum(lo8 + c * C, N_TOK - C)
                    pltpu.sync_copy(src_hbm.at[pl.ds(off, C)], ia)
                    pltpu.sync_copy(dst_hbm.at[pl.ds(off, C)], idd)
                    pltpu.async_copy(table_hbm.at[ia], rows, semg)

            def finish_chunk(c, idd, rows, semg, sems_):
                @pl.when(jnp.logical_and(c >= 0, c < nch))
                def _():
                    pltpu.make_async_copy(table_hbm.at[pl.ds(0, C)], rows,
                                          semg).wait()
                    pltpu.async_copy(rows, out_hbm.at[idd], sems_)

            def step(c, c2):
                @pl.when(c % 2 == 0)
                def _():
                    start_chunk(c, ia0, id0, rows0, semg0, sems0)
                    finish_chunk(c - 1, id1, rows1, semg1, sems1)

                @pl.when(c % 2 == 1)
                def _():
                    start_chunk(c, ia1, id1, rows1, semg1, sems1)
                    finish_chunk(c - 1, id0, rows0, semg0, sems0)

                return c2

            lax.fori_loop(0, nch + 1, step, 0)

            @pl.when(nch >= 1)
            def _():
                pltpu.make_async_copy(out_hbm.at[pl.ds(r0, C)], rows0,
                                      sems0).wait()

            @pl.when(nch >= 2)
            def _():
                pltpu.make_async_copy(out_hbm.at[pl.ds(r0, C)], rows1,
                                      sems1).wait()

        gather_scatter(t_lo, t_hi, tsrc_hbm, tdst_hbm, embed_hbm)
        gather_scatter(i_lo, i_hi, isrc_hbm, idst_hbm, imgf_hbm)
        return carry

    lax.fori_loop(0, 2, piece, 0)


def kernel(input_ids, image_mask, segment_ids, image_features, embed_table):
    ids = input_ids.astype(jnp.int32)
    seg = segment_ids.astype(jnp.int32)
    mask = image_mask.astype(jnp.int32)
    is_img = mask.astype(bool)
    ar = jnp.arange(N_TOK, dtype=jnp.int32)

    ipos = jnp.cumsum(mask, dtype=jnp.int32) - 1
    img_row = jnp.clip(ipos, 0, image_features.shape[0] - 1)
    starts = jnp.searchsorted(seg, jnp.arange(B, dtype=jnp.int32)).astype(jnp.int32)
    pos = jnp.clip(ar - starts[seg], 0, MAX_LEN - 1)
    dst = seg * MAX_LEN + pos

    tmask = 1 - mask
    tpos = jnp.cumsum(tmask, dtype=jnp.int32) - 1
    n_t = tpos[-1] + 1
    n_i = ipos[-1] + 1
    t_at = jnp.where(is_img, N_TOK, tpos)
    i_at = jnp.where(is_img, ipos, N_TOK)
    z = jnp.zeros(N_TOK + 1, jnp.int32)
    t_src = z.at[t_at].set(ids)[:N_TOK]
    t_dst = z.at[t_at].set(dst)[:N_TOK]
    i_src = z.at[i_at].set(img_row)[:N_TOK]
    i_dst = z.at[i_at].set(dst)[:N_TOK]

    t_last = jnp.clip(n_t - 1, 0, N_TOK - 1)
    i_last = jnp.clip(n_i - 1, 0, N_TOK - 1)
    t_src = jnp.where(ar < n_t, t_src, t_src[t_last])
    t_dst = jnp.where(ar < n_t, t_dst, t_dst[t_last])
    i_src = jnp.where(ar < n_i, i_src, i_src[i_last])
    i_dst = jnp.where(ar < n_i, i_dst, i_dst[i_last])

    r0s = jnp.arange(NP + 1, dtype=jnp.int32) * PRW
    t_bnd = jnp.minimum(jnp.searchsorted(t_dst, r0s), n_t).astype(jnp.int32)
    i_bnd = jnp.minimum(jnp.searchsorted(i_dst, r0s), n_i).astype(jnp.int32)
    lens = jnp.diff(jnp.append(starts, jnp.int32(N_TOK)))
    q = jnp.arange(NP, dtype=jnp.int32)
    pad_off = jnp.clip(lens[q // 4] - (q % 4) * PRW, 0, PRW).astype(jnp.int32)

    meta = jnp.zeros((NP, 16), jnp.int32)
    meta = (meta.at[:, 0].set(t_bnd[:-1]).at[:, 1].set(t_bnd[1:])
                .at[:, 2].set(i_bnd[:-1]).at[:, 3].set(i_bnd[1:])
                .at[:, 4].set(pad_off)).reshape(-1)

    mesh = plsc.VectorSubcoreMesh(core_axis_name="c", subcore_axis_name="s")
    run = functools.partial(
        pl.kernel,
        mesh=mesh,
        out_type=jax.ShapeDtypeStruct((B * MAX_LEN, D), jnp.float32),
        scratch_types=[
            pltpu.VMEM((NP * 16,), jnp.int32),
            pltpu.VMEM((C,), jnp.int32),
            pltpu.VMEM((C,), jnp.int32),
            pltpu.VMEM((C,), jnp.int32),
            pltpu.VMEM((C,), jnp.int32),
            pltpu.VMEM((C, D), jnp.float32),
            pltpu.VMEM((C, D), jnp.float32),
            pltpu.VMEM((ZC, D), jnp.float32),
            pltpu.SemaphoreType.DMA,
            pltpu.SemaphoreType.DMA,
            pltpu.SemaphoreType.DMA,
            pltpu.SemaphoreType.DMA,
            pltpu.SemaphoreType.DMA,
        ],
    )(_body)
    out = run(embed_table, image_features, t_src, t_dst, i_src, i_dst, meta)

    padded = out.reshape(B, MAX_LEN, D)
    attn = jnp.arange(MAX_LEN, dtype=jnp.int32)[None, :] < jnp.minimum(lens, MAX_LEN)[:, None]
    return padded, attn


# speedup vs baseline: 1.1026x; 1.1026x over previous
import functools

import jax
import jax.numpy as jnp
from jax import lax
from jax.experimental import pallas as pl
from jax.experimental.pallas import tpu as pltpu
from jax.experimental.pallas import tpu_sc as plsc

B = 16
MAX_LEN = 4096
D = 1024
N_TOK = 32768
NW = 32
RPW = (B * MAX_LEN) // NW
C = 48
ZC = 16
ZRING = 4


def _body(embed_hbm, imgf_hbm, tsrc_hbm, tdst_hbm, isrc_hbm, idst_hbm,
          meta_hbm, out_hbm, meta_v, ia0, id0, ia1, id1, rows0, rows1, zbuf,
          semz, semg0, semg1, sems0, sems1):
    cid = lax.axis_index("c")
    sid = lax.axis_index("s")
    wid = sid * 2 + cid
    r0 = wid * RPW

    pltpu.sync_copy(meta_hbm, meta_v)
    mvec = meta_v[pl.ds(wid * 16, 16)]
    t_lo = mvec[0]
    t_hi = mvec[1]
    i_lo = mvec[2]
    i_hi = mvec[3]
    pad_off = mvec[4]

    def zfill(k, carry):
        row = k // 8
        col0 = (k % 8) * 128
        for u in range(8):
            zbuf[row, pl.ds(col0 + u * 16, 16)] = jnp.zeros((16,), jnp.float32)
        return carry

    lax.fori_loop(0, (ZC * D) // 128, zfill, 0)

    p8 = (pad_off // 8) * 8
    nz = RPW - p8
    nfull = nz // ZC

    def zdrain():
        pltpu.make_async_copy(out_hbm.at[pl.ds(r0, ZC)], zbuf, semz).wait()

    def zchunk(j, carry):
        start = r0 + RPW - (j + 1) * ZC
        pltpu.async_copy(zbuf, out_hbm.at[pl.ds(start, ZC)], semz)

        @pl.when(j >= ZRING)
        def _():
            zdrain()

        return carry

    lax.fori_loop(0, nfull, zchunk, 0)

    def zdrain_loop(j, carry):
        zdrain()
        return carry

    lax.fori_loop(0, jnp.minimum(nfull, ZRING), zdrain_loop, 0)

    h = nz - nfull * ZC

    @pl.when(h == 8)
    def _():
        pltpu.sync_copy(zbuf.at[pl.ds(0, 8)], out_hbm.at[pl.ds(r0 + p8, 8)])

    def gather_scatter(lo, hi, src_hbm, dst_hbm, table_hbm):
        lo8 = (lo // 8) * 8
        n = hi - lo8
        nch = (n + C - 1) // C

        def chunk(c, ia, idd, rows, semg, sems_):
            @pl.when(c < nch)
            def _():
                @pl.when(c >= 2)
                def _():
                    pltpu.make_async_copy(out_hbm.at[pl.ds(r0, C)], rows,
                                          sems_).wait()

                off = jnp.minimum(lo8 + c * C, N_TOK - C)
                pltpu.sync_copy(src_hbm.at[pl.ds(off, C)], ia)
                pltpu.sync_copy(dst_hbm.at[pl.ds(off, C)], idd)
                pltpu.async_copy(table_hbm.at[ia], rows, semg).wait()
                pltpu.async_copy(rows, out_hbm.at[idd], sems_)

        def pair(c2, carry):
            chunk(2 * c2, ia0, id0, rows0, semg0, sems0)
            chunk(2 * c2 + 1, ia1, id1, rows1, semg1, sems1)
            return carry

        lax.fori_loop(0, (nch + 1) // 2, pair, 0)

        @pl.when(nch >= 1)
        def _():
            pltpu.make_async_copy(out_hbm.at[pl.ds(r0, C)], rows0, sems0).wait()

        @pl.when(nch >= 2)
        def _():
            pltpu.make_async_copy(out_hbm.at[pl.ds(r0, C)], rows1, sems1).wait()

    gather_scatter(t_lo, t_hi, tsrc_hbm, tdst_hbm, embed_hbm)
    gather_scatter(i_lo, i_hi, isrc_hbm, idst_hbm, imgf_hbm)


def kernel(input_ids, image_mask, segment_ids, image_features, embed_table):
    ids = input_ids.astype(jnp.int32)
    seg = segment_ids.astype(jnp.int32)
    mask = image_mask.astype(jnp.int32)
    is_img = mask.astype(bool)
    ar = jnp.arange(N_TOK, dtype=jnp.int32)

    ipos = jnp.cumsum(mask, dtype=jnp.int32) - 1
    img_row = jnp.clip(ipos, 0, image_features.shape[0] - 1)
    starts = jnp.searchsorted(seg, jnp.arange(B, dtype=jnp.int32)).astype(jnp.int32)
    pos = jnp.clip(ar - starts[seg], 0, MAX_LEN - 1)
    dst = seg * MAX_LEN + pos

    tmask = 1 - mask
    tpos = jnp.cumsum(tmask, dtype=jnp.int32) - 1
    n_t = tpos[-1] + 1
    n_i = ipos[-1] + 1
    t_at = jnp.where(is_img, N_TOK, tpos)
    i_at = jnp.where(is_img, ipos, N_TOK)
    z = jnp.zeros(N_TOK + 1, jnp.int32)
    t_src = z.at[t_at].set(ids)[:N_TOK]
    t_dst = z.at[t_at].set(dst)[:N_TOK]
    i_src = z.at[i_at].set(img_row)[:N_TOK]
    i_dst = z.at[i_at].set(dst)[:N_TOK]

    t_last = jnp.clip(n_t - 1, 0, N_TOK - 1)
    i_last = jnp.clip(n_i - 1, 0, N_TOK - 1)
    t_src = jnp.where(ar < n_t, t_src, t_src[t_last])
    t_dst = jnp.where(ar < n_t, t_dst, t_dst[t_last])
    i_src = jnp.where(ar < n_i, i_src, i_src[i_last])
    i_dst = jnp.where(ar < n_i, i_dst, i_dst[i_last])

    r0s = jnp.arange(NW + 1, dtype=jnp.int32) * RPW
    t_bnd = jnp.minimum(jnp.searchsorted(t_dst, r0s), n_t).astype(jnp.int32)
    i_bnd = jnp.minimum(jnp.searchsorted(i_dst, r0s), n_i).astype(jnp.int32)
    lens = jnp.diff(jnp.append(starts, jnp.int32(N_TOK)))
    w = jnp.arange(NW, dtype=jnp.int32)
    pad_off = jnp.clip(lens[w // 2] - (w % 2) * RPW, 0, RPW).astype(jnp.int32)

    meta = jnp.zeros((NW, 16), jnp.int32)
    meta = (meta.at[:, 0].set(t_bnd[:-1]).at[:, 1].set(t_bnd[1:])
                .at[:, 2].set(i_bnd[:-1]).at[:, 3].set(i_bnd[1:])
                .at[:, 4].set(pad_off)).reshape(-1)

    mesh = plsc.VectorSubcoreMesh(core_axis_name="c", subcore_axis_name="s")
    run = functools.partial(
        pl.kernel,
        mesh=mesh,
        out_type=jax.ShapeDtypeStruct((B * MAX_LEN, D), jnp.float32),
        scratch_types=[
            pltpu.VMEM((NW * 16,), jnp.int32),
            pltpu.VMEM((C,), jnp.int32),
            pltpu.VMEM((C,), jnp.int32),
            pltpu.VMEM((C,), jnp.int32),
            pltpu.VMEM((C,), jnp.int32),
            pltpu.VMEM((C, D), jnp.float32),
            pltpu.VMEM((C, D), jnp.float32),
            pltpu.VMEM((ZC, D), jnp.float32),
            pltpu.SemaphoreType.DMA,
            pltpu.SemaphoreType.DMA,
            pltpu.SemaphoreType.DMA,
            pltpu.SemaphoreType.DMA,
            pltpu.SemaphoreType.DMA,
        ],
    )(_body)
    out = run(embed_table, image_features, t_src, t_dst, i_src, i_dst, meta)

    padded = out.reshape(B, MAX_LEN, D)
    attn = jnp.arange(MAX_LEN, dtype=jnp.int32)[None, :] < jnp.minimum(lens, MAX_LEN)[:, None]
    return padded, attn
